# Initial kernel scaffold; baseline (speedup 1.0000x reference)
import functools

import jax
import jax.numpy as jnp
from jax import lax
from jax.experimental import pallas as pl
from jax.experimental.pallas import tpu as pltpu
from jax.experimental.pallas import tpu_sc as plsc

_N_NODES = 10000
_D = 128
_E = 320000
_K = 10
_ALPHA = 0.1

_NC, _NS = 2, 16
_NW = _NC * _NS
_EPW = _E // _NW
_CHUNK = 80
_NCHUNK = _EPW // _CHUNK
_ROWS_PER_TILE = _N_NODES // _NS

_mesh = plsc.VectorSubcoreMesh(core_axis_name="c", subcore_axis_name="s")


@functools.partial(
    pl.kernel,
    out_type=jax.ShapeDtypeStruct((_NC, _N_NODES, _D), jnp.float32),
    mesh=_mesh,
    scratch_types=[
        pltpu.VMEM_SHARED((_N_NODES, _D), jnp.float32),
        pltpu.VMEM((_NCHUNK, _CHUNK), jnp.int32),
        pltpu.VMEM((_NCHUNK, _CHUNK), jnp.int32),
        pltpu.VMEM((_EPW,), jnp.float32),
        pltpu.VMEM((_CHUNK, _D), jnp.float32),
    ],
)
def _sc_propagate(h_hbm, src_hbm, dst_hbm, w_hbm, z_hbm, out_hbm,
                  acc, srcb, dstb, wb, msg):
    cid = lax.axis_index("c")
    sid = lax.axis_index("s")
    wid = sid * _NC + cid

    pltpu.sync_copy(src_hbm.at[wid], srcb)
    pltpu.sync_copy(dst_hbm.at[wid], dstb)
    pltpu.sync_copy(w_hbm.at[wid], wb)

    pltpu.sync_copy(z_hbm, acc.at[pl.ds(sid * _ROWS_PER_TILE, _ROWS_PER_TILE)])
    plsc.subcore_barrier()

    @pl.loop(0, _NCHUNK)
    def _chunk_loop(ci):
        pltpu.sync_copy(h_hbm.at[srcb.at[ci]], msg)

        @pl.loop(0, _CHUNK)
        def _scale(e):
            widx = jnp.full((16,), ci * _CHUNK + e, jnp.int32)
            wv = plsc.load_gather(wb, [widx])
            for j in range(_D // 16):
                sl = pl.ds(j * 16, 16)
                msg[e, sl] = msg[e, sl] * wv

        pltpu.sync_copy(msg, acc.at[dstb.at[ci]], add=True)

    plsc.subcore_barrier()
    rows = pl.ds(sid * _ROWS_PER_TILE, _ROWS_PER_TILE)
    pltpu.sync_copy(acc.at[rows], out_hbm.at[cid, rows])


def _combine_body(a_ref, b_ref, f_ref, o_ref):
    o_ref[...] = (1.0 - _ALPHA) * (a_ref[...] + b_ref[...]) + _ALPHA * f_ref[...]


_combine = pl.pallas_call(
    _combine_body,
    grid=(25,),
    in_specs=[pl.BlockSpec((400, _D), lambda i: (i, 0))] * 3,
    out_specs=pl.BlockSpec((400, _D), lambda i: (i, 0)),
    out_shape=jax.ShapeDtypeStruct((_N_NODES, _D), jnp.float32),
)


@jax.jit
def kernel(feat, edge_index, edge_weight):
    src = edge_index[0].reshape(_NW, _NCHUNK, _CHUNK)
    dst = edge_index[1].reshape(_NW, _NCHUNK, _CHUNK)
    w = edge_weight.reshape(_NW, _EPW)
    zrows = jnp.zeros((_ROWS_PER_TILE, _D), jnp.float32)
    h = feat
    for _ in range(_K):
        acc = _sc_propagate(h, src, dst, w, zrows)
        h = _combine(acc[0], acc[1], feat)
    return h


# speedup vs baseline: 4.5691x; 4.5691x over previous
import functools

import jax
import jax.numpy as jnp
from jax import lax
from jax.experimental import pallas as pl
from jax.experimental.pallas import tpu as pltpu
from jax.experimental.pallas import tpu_sc as plsc

_N_NODES = 10000
_D = 128
_E = 320000
_K = 10
_ALPHA = 0.1

_NC, _NS = 2, 16
_NW = _NC * _NS
_EPW = _E // _NW
_CHUNK = 80
_NPHASE = 5
_CPP = 25
_EPP = _CPP * _CHUNK
_N_PAD = 10240
_ROWS_PER_TILE = _N_PAD // _NS

_mesh = plsc.VectorSubcoreMesh(core_axis_name="c", subcore_axis_name="s")


@functools.partial(
    pl.kernel,
    out_type=jax.ShapeDtypeStruct((_NC, _N_PAD, _D), jnp.float32),
    mesh=_mesh,
    scratch_types=[
        pltpu.VMEM_SHARED((_N_PAD, _D), jnp.float32),
        pltpu.VMEM((_CPP, _CHUNK), jnp.int32),
        pltpu.VMEM((_CPP, _CHUNK), jnp.int32),
        pltpu.VMEM((_EPP,), jnp.float32),
        pltpu.VMEM((_CHUNK, _D), jnp.float32),
    ],
    compiler_params=pltpu.CompilerParams(needs_layout_passes=False),
)
def _sc_propagate(h_hbm, src_hbm, dst_hbm, w_hbm, z_hbm, out_hbm,
                  acc, srcb, dstb, wb, msg):
    cid = lax.axis_index("c")
    sid = lax.axis_index("s")
    wid = sid * _NC + cid

    pltpu.sync_copy(z_hbm, acc.at[pl.ds(sid * _ROWS_PER_TILE, _ROWS_PER_TILE)])
    plsc.subcore_barrier()

    for p in range(_NPHASE):
        pltpu.sync_copy(src_hbm.at[wid * _NPHASE + p], srcb)
        pltpu.sync_copy(dst_hbm.at[wid * _NPHASE + p], dstb)
        pltpu.sync_copy(w_hbm.at[wid * _NPHASE + p], wb)

        @pl.loop(0, _CPP)
        def _chunk_loop(ci):
            pltpu.sync_copy(h_hbm.at[srcb.at[ci]], msg)

            @pl.loop(0, _CHUNK)
            def _scale(e):
                widx = jnp.full((16,), ci * _CHUNK + e, jnp.int32)
                wv = plsc.load_gather(wb, [widx])
                for j in range(_D // 16):
                    sl = pl.ds(j * 16, 16)
                    msg[e, sl] = msg[e, sl] * wv

            pltpu.sync_copy(msg, acc.at[dstb.at[ci]], add=True)

    plsc.subcore_barrier()
    rows = pl.ds(sid * _ROWS_PER_TILE, _ROWS_PER_TILE)
    pltpu.sync_copy(acc.at[rows], out_hbm.at[cid, rows])


def _combine_body(a_ref, b_ref, f_ref, o_ref):
    o_ref[...] = (1.0 - _ALPHA) * (a_ref[...] + b_ref[...]) + _ALPHA * f_ref[...]


_combine = pl.pallas_call(
    _combine_body,
    grid=(_N_PAD // 640,),
    in_specs=[pl.BlockSpec((640, _D), lambda i: (i, 0))] * 3,
    out_specs=pl.BlockSpec((640, _D), lambda i: (i, 0)),
    out_shape=jax.ShapeDtypeStruct((_N_PAD, _D), jnp.float32),
)


@jax.jit
def kernel(feat, edge_index, edge_weight):
    src = edge_index[0].reshape(_NW * _NPHASE, _CPP, _CHUNK)
    dst = edge_index[1].reshape(_NW * _NPHASE, _CPP, _CHUNK)
    w = edge_weight.reshape(_NW * _NPHASE, _EPP)
    zrows = jnp.zeros((_ROWS_PER_TILE, _D), jnp.float32)
    feat_pad = jnp.concatenate(
        [feat, jnp.zeros((_N_PAD - _N_NODES, _D), jnp.float32)])
    h = feat_pad
    for _ in range(_K):
        acc = _sc_propagate(h, src, dst, w, zrows)
        h = _combine(acc[0], acc[1], feat_pad)
    return h[:_N_NODES]
